# baseline (speedup 1.0000x reference)
import jax
import jax.numpy as jnp
from jax import lax
from jax.experimental import pallas as pl
from jax.experimental.pallas import tpu as pltpu
from jax.experimental.pallas import tpu_sc as plsc

D = 128
NROWS = 100
N = 100000
NC, NS = 2, 16
NW = NC * NS
BPW = N // NW
CHUNK = 125
CPW = BPW // CHUNK
NBUF = 5
NOUT = CPW // NBUF
NLANE = 16
GRP = 128
ISTAGE = BPW + 11
ZPAD = 8


def _emb_body(z_hbm, tab_hbm, out_hbm, tab_v, idx_v, stage, wsems):
    wid = lax.axis_index("s") * NC + lax.axis_index("c")
    base = wid * BPW
    astart = (base // 8) * 8
    s = base - astart
    pltpu.sync_copy(tab_hbm, tab_v)
    pltpu.sync_copy(z_hbm.at[pl.ds(astart, ISTAGE)], idx_v)

    CW = CHUNK * D

    def write(j):
        boff = (j % NBUF) * CW
        return pltpu.make_async_copy(
            stage.at[pl.ds(boff, CW)],
            out_hbm.at[pl.ds((base + j * CHUNK) * D, CW)],
            wsems.at[j % NBUF])

    lanes = jax.lax.iota(jnp.int32, NLANE)
    lconst = lanes * D
    rots = [(lanes + d) & (NLANE - 1) for d in range(NLANE)]

    def chunk(j, carry):
        boff = (j % NBUF) * CW

        @pl.when(j >= NBUF)
        def _():
            write(j - NBUF).wait()

        @plsc.parallel_loop(0, GRP, step=NLANE)
        def _group(i):
            zv = idx_v[pl.ds(s + j * CHUNK + i, NLANE)]
            offv = zv * D
            valid = lanes < (CHUNK - i)
            for c in range(D // NLANE):
                offc = offv + c * NLANE
                dstc = lconst + (boff + i * D + c * NLANE)
                for d in range(NLANE):
                    g = plsc.load_gather(tab_v, [offc + rots[d]])
                    plsc.store_scatter(stage, [dstc + rots[d]], g, mask=valid)

        write(j).start()
        return carry

    lax.fori_loop(0, CPW, chunk, 0)
    for j in range(CPW - NBUF, CPW):
        write(j).wait()


@jax.jit
def _emb(z1d, tab_flat):
    f = pl.kernel(
        _emb_body,
        out_type=jax.ShapeDtypeStruct((N * D,), jnp.float32),
        mesh=plsc.VectorSubcoreMesh(core_axis_name="c", subcore_axis_name="s"),
        compiler_params=pltpu.CompilerParams(needs_layout_passes=False),
        scratch_types=[
            pltpu.VMEM((NROWS * D,), jnp.float32),
            pltpu.VMEM((ISTAGE,), jnp.int32),
            pltpu.VMEM((NBUF * CHUNK * D,), jnp.float32),
            pltpu.SemaphoreType.DMA((NBUF,)),
        ],
    )
    return f(z1d, tab_flat)


def kernel(Z, emb_weight):
    z = jnp.pad(Z.astype(jnp.int32), (0, ZPAD))
    out = _emb(z, emb_weight.reshape(-1))
    return out.reshape(N, D)


# speedup vs baseline: 2.2072x; 2.2072x over previous
import jax
import jax.numpy as jnp
from jax import lax
from jax.experimental import pallas as pl
from jax.experimental.pallas import tpu as pltpu
from jax.experimental.pallas import tpu_sc as plsc

D = 128
NROWS = 100
N = 100000
NC, NS = 2, 16
NW = NC * NS
BPW = N // NW
CHUNK = 125
CPW = BPW // CHUNK
NBUF = 5
NOUT = CPW // NBUF
NLANE = 16
GRP = 128
ISTAGE = BPW + 11
ZPAD = 8


def _emb_body(z_hbm, tab_hbm, out_hbm, tab_v, idx_v, stage, wsems):
    wid = lax.axis_index("s") * NC + lax.axis_index("c")
    base = wid * BPW
    astart = (base // 8) * 8
    s = base - astart
    pltpu.sync_copy(tab_hbm, tab_v)
    pltpu.sync_copy(z_hbm.at[pl.ds(astart, ISTAGE)], idx_v)

    CW = CHUNK * D
    SLOT = GRP * D

    def write(j):
        boff = (j % NBUF) * SLOT
        return pltpu.make_async_copy(
            stage.at[pl.ds(boff, CW)],
            out_hbm.at[pl.ds((base + j * CHUNK) * D, CW)],
            wsems.at[j % NBUF])

    lanes = jax.lax.iota(jnp.int32, NLANE)
    lconst = lanes * D
    rots = [(lanes + d) & (NLANE - 1) for d in range(NLANE)]

    def chunk(j, carry):
        boff = (j % NBUF) * SLOT

        @pl.when(j >= NBUF)
        def _():
            write(j - NBUF).wait()

        @plsc.parallel_loop(0, GRP, step=NLANE)
        def _group(i):
            zv = idx_v[pl.ds(s + j * CHUNK + i, NLANE)]
            for k in range(NLANE):
                off = zv[k] * D
                dst = boff + i * D + k * D
                for c in range(D // NLANE):
                    stage[pl.ds(dst + c * NLANE, NLANE)] = (
                        tab_v[pl.ds(off + c * NLANE, NLANE)])

        write(j).start()
        return carry

    lax.fori_loop(0, CPW, chunk, 0)
    for j in range(CPW - NBUF, CPW):
        write(j).wait()


@jax.jit
def _emb(z1d, tab_flat):
    f = pl.kernel(
        _emb_body,
        out_type=jax.ShapeDtypeStruct((N * D,), jnp.float32),
        mesh=plsc.VectorSubcoreMesh(core_axis_name="c", subcore_axis_name="s"),
        compiler_params=pltpu.CompilerParams(needs_layout_passes=False),
        scratch_types=[
            pltpu.VMEM((NROWS * D,), jnp.float32),
            pltpu.VMEM((ISTAGE,), jnp.int32),
            pltpu.VMEM((NBUF * GRP * D,), jnp.float32),
            pltpu.SemaphoreType.DMA((NBUF,)),
        ],
    )
    return f(z1d, tab_flat)


def kernel(Z, emb_weight):
    z = jnp.pad(Z.astype(jnp.int32), (0, ZPAD))
    out = _emb(z, emb_weight.reshape(-1))
    return out.reshape(N, D)
